# Initial kernel scaffold; baseline (speedup 1.0000x reference)
import jax
import jax.numpy as jnp
from jax.experimental import pallas as pl


def kernel(x, edge_index, W1, a_src1, a_dst1, b1, W2, a_src2, a_dst2, b2):
    raise NotImplementedError("write your pallas kernel here")


# speedup vs baseline: 21.7647x; 21.7647x over previous
import dataclasses
import functools

import jax
import jax.numpy as jnp
from jax import lax
from jax.experimental import pallas as pl
from jax.experimental.pallas import tpu as pltpu
from jax.experimental.pallas import tpu_sc as plsc

H = 4
NEG_SLOPE = 0.2

NC = 2
NS = 16
NW = NC * NS
LANES = 16

CHA = 400
CHB = 40
CHL = 400


def _sc_params():
    cp = pltpu.CompilerParams()
    if "needs_layout_passes" in pltpu.CompilerParams.__dataclass_fields__:
        cp = dataclasses.replace(cp, needs_layout_passes=False)
    return cp


def _feats_kernel(nb, d, hc):
    def body(x_ref, w_ref, a_ref, h_ref, alph_ref, m_ref):
        i = pl.program_id(0)
        h = jnp.dot(x_ref[...], w_ref[...], preferred_element_type=jnp.float32)
        h_ref[...] = h
        alph = jnp.dot(h, a_ref[...], preferred_element_type=jnp.float32)
        alph_ref[...] = alph
        bmax = jnp.broadcast_to(jnp.max(alph, axis=0, keepdims=True), (8, 128))

        @pl.when(i == 0)
        def _():
            m_ref[...] = bmax

        @pl.when(i != 0)
        def _():
            m_ref[...] = jnp.maximum(m_ref[...], bmax)

    return body


def _tc_feats(xin, w, amat):
    n, d = xin.shape
    nb = n // 8
    hc = w.shape[1]
    grid = (n // nb,)
    return pl.pallas_call(
        _feats_kernel(nb, d, hc),
        grid=grid,
        in_specs=[
            pl.BlockSpec((nb, d), lambda i: (i, 0)),
            pl.BlockSpec((d, hc), lambda i: (0, 0)),
            pl.BlockSpec((hc, 128), lambda i: (0, 0)),
        ],
        out_specs=[
            pl.BlockSpec((nb, hc), lambda i: (i, 0)),
            pl.BlockSpec((nb, 128), lambda i: (i, 0)),
            pl.BlockSpec((8, 128), lambda i: (0, 0)),
        ],
        out_shape=[
            jax.ShapeDtypeStruct((n, hc), jnp.float32),
            jax.ShapeDtypeStruct((n, 128), jnp.float32),
            jax.ShapeDtypeStruct((8, 128), jnp.float32),
        ],
    )(xin, w, amat)


def _tc_invd(dpart):
    n4 = dpart.shape[1]

    def body(d_ref, o_ref):
        s = jnp.sum(d_ref[...], axis=0)
        o_ref[...] = 0.25 / (s + 1e-16)

    return pl.pallas_call(
        body,
        out_shape=jax.ShapeDtypeStruct((n4,), jnp.float32),
    )(dpart)


def _tc_mid(outp, b):
    n = outp.shape[1]

    def body(p_ref, b_ref, o_ref):
        v = p_ref[0] + p_ref[1] + b_ref[...]
        o_ref[...] = jnp.where(v > 0, v, jnp.exp(v) - 1.0)

    return pl.pallas_call(
        body,
        out_shape=jax.ShapeDtypeStruct((n, 128), jnp.float32),
    )(outp, b)


def _tc_fin(outp, b):
    n = outp.shape[1]

    def body(p_ref, b_ref, o_ref):
        o_ref[...] = p_ref[0] + p_ref[1] + b_ref[...]

    return pl.pallas_call(
        body,
        out_shape=jax.ShapeDtypeStruct((n, 128), jnp.float32),
    )(outp, b)


def _sc_pass_a(as_flat, ad_flat, mvec, edge_flat):
    n4 = as_flat.shape[0]
    e = edge_flat.shape[0] // 2
    et = e // NW
    nch = et // CHA

    mesh = plsc.VectorSubcoreMesh(
        core_axis_name="c", subcore_axis_name="s", num_cores=NC,
        num_subcores=NS)

    @functools.partial(
        pl.kernel,
        out_type=(
            jax.ShapeDtypeStruct((4 * e,), jnp.float32),
            jax.ShapeDtypeStruct((NW, n4), jnp.float32),
        ),
        mesh=mesh,
        compiler_params=_sc_params(),
        scratch_types=[
            pltpu.VMEM((n4,), jnp.float32),
            pltpu.VMEM((n4,), jnp.float32),
            pltpu.VMEM((n4,), jnp.float32),
            pltpu.VMEM((CHA,), jnp.int32),
            pltpu.VMEM((CHA,), jnp.int32),
            pltpu.VMEM((4 * CHA,), jnp.float32),
            pltpu.VMEM((LANES,), jnp.float32),
        ],
    )
    def kern(as_hbm, ad_hbm, m_hbm, ei_hbm, w_hbm, dp_hbm,
             as_t, ad_t, den_t, src_c, dst_c, w_c, m_t):
        cid = lax.axis_index("c")
        sid = lax.axis_index("s")
        wid = cid * NS + sid
        base = wid * et

        pltpu.sync_copy(as_hbm, as_t)
        pltpu.sync_copy(ad_hbm, ad_t)
        pltpu.sync_copy(m_hbm, m_t)

        @pl.loop(0, n4, step=LANES)
        def _(i):
            den_t[pl.ds(i, LANES)] = jnp.zeros((LANES,), jnp.float32)

        iota = lax.iota(jnp.int32, LANES)
        hv = iota & 3
        ev = iota >> 2
        mv = m_t[...]

        @pl.loop(0, nch)
        def _(g):
            off = base + g * CHA
            pltpu.sync_copy(ei_hbm.at[pl.ds(off, CHA)], src_c)
            pltpu.sync_copy(ei_hbm.at[pl.ds(e + off, CHA)], dst_c)

            @pl.loop(0, CHA // 4)
            def _(k):
                eidx = k * 4 + ev
                srcv = plsc.load_gather(src_c, [eidx])
                dstv = plsc.load_gather(dst_c, [eidx])
                asv = plsc.load_gather(as_t, [srcv * 4 + hv])
                adv = plsc.load_gather(ad_t, [dstv * 4 + hv])
                s = asv + adv
                s = jnp.maximum(s, NEG_SLOPE * s)
                w = jnp.exp(s - mv)
                w_c[pl.ds(k * LANES, LANES)] = w
                plsc.addupdate_scatter(den_t, [dstv * 4 + hv], w)

            pltpu.sync_copy(w_c, w_hbm.at[pl.ds(off * 4, 4 * CHA)])

        pltpu.sync_copy(den_t, dp_hbm.at[wid])

    return kern(as_flat, ad_flat, mvec, edge_flat)


def _sc_alpha(w_flat, invd, edge_flat):
    n4 = invd.shape[0]
    e = edge_flat.shape[0] // 2
    et = e // NW
    nch = et // CHL

    mesh = plsc.VectorSubcoreMesh(
        core_axis_name="c", subcore_axis_name="s", num_cores=NC,
        num_subcores=NS)

    @functools.partial(
        pl.kernel,
        out_type=jax.ShapeDtypeStruct((4 * e,), jnp.float32),
        mesh=mesh,
        compiler_params=_sc_params(),
        scratch_types=[
            pltpu.VMEM((n4,), jnp.float32),
            pltpu.VMEM((CHL,), jnp.int32),
            pltpu.VMEM((4 * CHL,), jnp.float32),
        ],
    )
    def kern(w_hbm, iv_hbm, ei_hbm, al_hbm, iv_t, dst_c, w_c):
        cid = lax.axis_index("c")
        sid = lax.axis_index("s")
        wid = cid * NS + sid
        base = wid * et

        pltpu.sync_copy(iv_hbm, iv_t)
        iota = lax.iota(jnp.int32, LANES)
        hv = iota & 3
        ev = iota >> 2

        @pl.loop(0, nch)
        def _(g):
            off = base + g * CHL
            pltpu.sync_copy(ei_hbm.at[pl.ds(e + off, CHL)], dst_c)
            pltpu.sync_copy(w_hbm.at[pl.ds(off * 4, 4 * CHL)], w_c)

            @pl.loop(0, CHL // 4)
            def _(k):
                eidx = k * 4 + ev
                dstv = plsc.load_gather(dst_c, [eidx])
                dv = plsc.load_gather(iv_t, [dstv * 4 + hv])
                wv = w_c[pl.ds(k * LANES, LANES)]
                w_c[pl.ds(k * LANES, LANES)] = wv * dv

            pltpu.sync_copy(w_c, al_hbm.at[pl.ds(off * 4, 4 * CHL)])

    return kern(w_flat, invd, edge_flat)


def _sc_pass_b(h, al_flat, edge_flat):
    n = h.shape[0]
    e = edge_flat.shape[0] // 2
    et = e // NW
    nch = et // CHB
    rows_per_tile = n // NS

    mesh = plsc.VectorSubcoreMesh(
        core_axis_name="c", subcore_axis_name="s", num_cores=NC,
        num_subcores=NS)

    @functools.partial(
        pl.kernel,
        out_type=jax.ShapeDtypeStruct((NC, n, 128), jnp.float32),
        mesh=mesh,
        compiler_params=_sc_params(),
        scratch_types=[
            pltpu.VMEM((CHB, 512), jnp.float32),
            pltpu.VMEM((CHB, 128), jnp.float32),
            pltpu.VMEM((CHB,), jnp.int32),
            pltpu.VMEM((CHB,), jnp.int32),
            pltpu.VMEM((4 * CHB,), jnp.float32),
            pltpu.VMEM_SHARED((n, 128), jnp.float32),
            pltpu.SemaphoreType.DMA,
        ],
    )
    def kern(h_hbm, al_hbm, ei_hbm, out_hbm,
             rows, comb, src_c, dst_c, wa, acc_s, sem):
        cid = lax.axis_index("c")
        sid = lax.axis_index("s")
        wid = cid * NS + sid
        base = wid * et

        @pl.loop(0, CHB)
        def _(r):
            for v in range(8):
                comb[r, pl.ds(v * LANES, LANES)] = jnp.zeros(
                    (LANES,), jnp.float32)

        zoff = sid * rows_per_tile
        nfull = rows_per_tile // CHB
        for j in range(nfull):
            pltpu.sync_copy(comb, acc_s.at[pl.ds(zoff + j * CHB, CHB)])
        rem = rows_per_tile - nfull * CHB
        if rem:
            pltpu.sync_copy(comb.at[pl.ds(0, rem)],
                            acc_s.at[pl.ds(zoff + nfull * CHB, rem)])
        plsc.subcore_barrier()

        @pl.loop(0, nch)
        def _(g):
            off = base + g * CHB
            pltpu.sync_copy(ei_hbm.at[pl.ds(off, CHB)], src_c)
            pltpu.sync_copy(ei_hbm.at[pl.ds(e + off, CHB)], dst_c)
            pltpu.sync_copy(al_hbm.at[pl.ds(off * 4, 4 * CHB)], wa)

            pltpu.async_copy(h_hbm.at[src_c], rows, sem).wait()

            @pl.loop(0, CHB)
            def _(ed):
                e4 = ed * 4
                acc = [None] * 8
                for hh in range(H):
                    av = plsc.load_gather(
                        wa, [jnp.full((LANES,), e4 + hh, jnp.int32)])
                    for v in range(8):
                        seg = rows[ed, pl.ds(hh * 128 + v * LANES, LANES)]
                        t = av * seg
                        acc[v] = t if hh == 0 else acc[v] + t
                for v in range(8):
                    comb[ed, pl.ds(v * LANES, LANES)] = acc[v]

            pltpu.sync_copy(comb, acc_s.at[dst_c], add=True)

        plsc.subcore_barrier()
        pltpu.sync_copy(
            acc_s.at[pl.ds(zoff, rows_per_tile)],
            out_hbm.at[cid, pl.ds(zoff, rows_per_tile)])

    return kern(h, al_flat, edge_flat)


def _pad_rows(a, np_):
    return jnp.pad(a, ((0, np_ - a.shape[0]),) + ((0, 0),) * (a.ndim - 1))


def _amat(a_src, a_dst):
    hh, c = a_src.shape
    eye = jnp.eye(hh, dtype=jnp.float32)
    s = jnp.einsum("hc,hk->hck", a_src, eye).reshape(hh * c, hh)
    d = jnp.einsum("hc,hk->hck", a_dst, eye).reshape(hh * c, hh)
    pad = jnp.zeros((hh * c, 128 - 2 * hh), jnp.float32)
    return jnp.concatenate([s, d, pad], axis=1)


def _gat_layer(xin, edge_flat, w, amat):
    n = xin.shape[0]
    h, alph, mrow = _tc_feats(xin, w, amat)
    as_flat = alph[:, 0:H].reshape(-1)
    ad_flat = alph[:, H:2 * H].reshape(-1)
    msum = mrow[0, 0:H] + mrow[0, H:2 * H]
    shift = jnp.maximum(msum, NEG_SLOPE * msum)
    mvec = jnp.tile(shift, LANES // H)
    w_flat, dpart = _sc_pass_a(as_flat, ad_flat, mvec, edge_flat)
    invd = _tc_invd(dpart)
    al_flat = _sc_alpha(w_flat, invd, edge_flat)
    return _sc_pass_b(h, al_flat, edge_flat)


def kernel(x, edge_index, W1, a_src1, a_dst1, b1, W2, a_src2, a_dst2, b2):
    n = x.shape[0]
    np_ = ((n + 8 * NS - 1) // (8 * NS)) * (8 * NS)
    edge_flat = edge_index.reshape(-1)
    outp1 = _gat_layer(_pad_rows(x, np_), edge_flat, W1, _amat(a_src1, a_dst1))
    hmid = _tc_mid(outp1, b1.reshape(1, -1))
    outp2 = _gat_layer(hmid, edge_flat, W2, _amat(a_src2, a_dst2))
    out = _tc_fin(outp2, b2.reshape(1, -1))
    return out[:n]


# speedup vs baseline: 27.5695x; 1.2667x over previous
import dataclasses
import functools

import jax
import jax.numpy as jnp
from jax import lax
from jax.experimental import pallas as pl
from jax.experimental.pallas import tpu as pltpu
from jax.experimental.pallas import tpu_sc as plsc

H = 4
NEG_SLOPE = 0.2

NC = 2
NS = 16
NW = NC * NS
LANES = 16

CHA = 400
CHB = 32
SCH = 256
CHL = 400


def _sc_params():
    cp = pltpu.CompilerParams()
    if "needs_layout_passes" in pltpu.CompilerParams.__dataclass_fields__:
        cp = dataclasses.replace(cp, needs_layout_passes=False)
    return cp


def _feats_kernel(nb, d, hc):
    def body(x_ref, w_ref, a_ref, h_ref, alph_ref, m_ref):
        i = pl.program_id(0)
        h = jnp.dot(x_ref[...], w_ref[...], preferred_element_type=jnp.float32)
        h_ref[...] = h
        alph = jnp.dot(h, a_ref[...], preferred_element_type=jnp.float32)
        alph_ref[...] = alph
        bmax = jnp.broadcast_to(jnp.max(alph, axis=0, keepdims=True), (8, 128))

        @pl.when(i == 0)
        def _():
            m_ref[...] = bmax

        @pl.when(i != 0)
        def _():
            m_ref[...] = jnp.maximum(m_ref[...], bmax)

    return body


def _tc_feats(xin, w, amat):
    n, d = xin.shape
    nb = n // 8
    hc = w.shape[1]
    grid = (n // nb,)
    return pl.pallas_call(
        _feats_kernel(nb, d, hc),
        grid=grid,
        in_specs=[
            pl.BlockSpec((nb, d), lambda i: (i, 0)),
            pl.BlockSpec((d, hc), lambda i: (0, 0)),
            pl.BlockSpec((hc, 128), lambda i: (0, 0)),
        ],
        out_specs=[
            pl.BlockSpec((nb, hc), lambda i: (i, 0)),
            pl.BlockSpec((nb, 128), lambda i: (i, 0)),
            pl.BlockSpec((8, 128), lambda i: (0, 0)),
        ],
        out_shape=[
            jax.ShapeDtypeStruct((n, hc), jnp.float32),
            jax.ShapeDtypeStruct((n, 128), jnp.float32),
            jax.ShapeDtypeStruct((8, 128), jnp.float32),
        ],
    )(xin, w, amat)


def _tc_invd(dpart):
    n4 = dpart.shape[1]

    def body(d_ref, o_ref):
        s = jnp.sum(d_ref[...], axis=0)
        o_ref[...] = 0.25 / (s + 1e-16)

    return pl.pallas_call(
        body,
        out_shape=jax.ShapeDtypeStruct((n4,), jnp.float32),
    )(dpart)


def _tc_mid(outp, b):
    n = outp.shape[1]

    def body(p_ref, b_ref, o_ref):
        v = p_ref[0] + p_ref[1] + b_ref[...]
        o_ref[...] = jnp.where(v > 0, v, jnp.exp(v) - 1.0)

    return pl.pallas_call(
        body,
        out_shape=jax.ShapeDtypeStruct((n, 128), jnp.float32),
    )(outp, b)


def _tc_fin(outp, b):
    n = outp.shape[1]

    def body(p_ref, b_ref, o_ref):
        o_ref[...] = p_ref[0] + p_ref[1] + b_ref[...]

    return pl.pallas_call(
        body,
        out_shape=jax.ShapeDtypeStruct((n, 128), jnp.float32),
    )(outp, b)


def _sc_pass_a(as_flat, ad_flat, mvec, edge_flat):
    n4 = as_flat.shape[0]
    e = edge_flat.shape[0] // 2
    et = e // NW
    nch = et // CHA

    mesh = plsc.VectorSubcoreMesh(
        core_axis_name="c", subcore_axis_name="s", num_cores=NC,
        num_subcores=NS)

    @functools.partial(
        pl.kernel,
        out_type=(
            jax.ShapeDtypeStruct((4 * e,), jnp.float32),
            jax.ShapeDtypeStruct((NW, n4), jnp.float32),
        ),
        mesh=mesh,
        compiler_params=_sc_params(),
        scratch_types=[
            pltpu.VMEM((n4,), jnp.float32),
            pltpu.VMEM((n4,), jnp.float32),
            pltpu.VMEM((n4,), jnp.float32),
            pltpu.VMEM((CHA,), jnp.int32),
            pltpu.VMEM((CHA,), jnp.int32),
            pltpu.VMEM((4 * CHA,), jnp.float32),
            pltpu.VMEM((LANES,), jnp.float32),
        ],
    )
    def kern(as_hbm, ad_hbm, m_hbm, ei_hbm, w_hbm, dp_hbm,
             as_t, ad_t, den_t, src_c, dst_c, w_c, m_t):
        cid = lax.axis_index("c")
        sid = lax.axis_index("s")
        wid = cid * NS + sid
        base = wid * et

        pltpu.sync_copy(as_hbm, as_t)
        pltpu.sync_copy(ad_hbm, ad_t)
        pltpu.sync_copy(m_hbm, m_t)

        @pl.loop(0, n4, step=LANES)
        def _(i):
            den_t[pl.ds(i, LANES)] = jnp.zeros((LANES,), jnp.float32)

        iota = lax.iota(jnp.int32, LANES)
        hv = iota & 3
        ev = iota >> 2
        mv = m_t[...]

        @pl.loop(0, nch)
        def _(g):
            off = base + g * CHA
            pltpu.sync_copy(ei_hbm.at[pl.ds(off, CHA)], src_c)
            pltpu.sync_copy(ei_hbm.at[pl.ds(e + off, CHA)], dst_c)

            @pl.loop(0, CHA // 4)
            def _(k):
                eidx = k * 4 + ev
                srcv = plsc.load_gather(src_c, [eidx])
                dstv = plsc.load_gather(dst_c, [eidx])
                asv = plsc.load_gather(as_t, [srcv * 4 + hv])
                adv = plsc.load_gather(ad_t, [dstv * 4 + hv])
                s = asv + adv
                s = jnp.maximum(s, NEG_SLOPE * s)
                w = jnp.exp(s - mv)
                w_c[pl.ds(k * LANES, LANES)] = w
                plsc.addupdate_scatter(den_t, [dstv * 4 + hv], w)

            pltpu.sync_copy(w_c, w_hbm.at[pl.ds(off * 4, 4 * CHA)])

        pltpu.sync_copy(den_t, dp_hbm.at[wid])

    return kern(as_flat, ad_flat, mvec, edge_flat)


def _sc_alpha(w_flat, invd, edge_flat):
    n4 = invd.shape[0]
    e = edge_flat.shape[0] // 2
    et = e // NW
    nch = et // CHL

    mesh = plsc.VectorSubcoreMesh(
        core_axis_name="c", subcore_axis_name="s", num_cores=NC,
        num_subcores=NS)

    @functools.partial(
        pl.kernel,
        out_type=jax.ShapeDtypeStruct((4 * e,), jnp.float32),
        mesh=mesh,
        compiler_params=_sc_params(),
        scratch_types=[
            pltpu.VMEM((n4,), jnp.float32),
            pltpu.VMEM((CHL,), jnp.int32),
            pltpu.VMEM((4 * CHL,), jnp.float32),
        ],
    )
    def kern(w_hbm, iv_hbm, ei_hbm, al_hbm, iv_t, dst_c, w_c):
        cid = lax.axis_index("c")
        sid = lax.axis_index("s")
        wid = cid * NS + sid
        base = wid * et

        pltpu.sync_copy(iv_hbm, iv_t)
        iota = lax.iota(jnp.int32, LANES)
        hv = iota & 3
        ev = iota >> 2

        @pl.loop(0, nch)
        def _(g):
            off = base + g * CHL
            pltpu.sync_copy(ei_hbm.at[pl.ds(e + off, CHL)], dst_c)
            pltpu.sync_copy(w_hbm.at[pl.ds(off * 4, 4 * CHL)], w_c)

            @pl.loop(0, CHL // 4)
            def _(k):
                eidx = k * 4 + ev
                dstv = plsc.load_gather(dst_c, [eidx])
                dv = plsc.load_gather(iv_t, [dstv * 4 + hv])
                wv = w_c[pl.ds(k * LANES, LANES)]
                w_c[pl.ds(k * LANES, LANES)] = wv * dv

            pltpu.sync_copy(w_c, al_hbm.at[pl.ds(off * 4, 4 * CHL)])

    return kern(w_flat, invd, edge_flat)


def _sc_pass_b(h, al_flat, src2d, dst2d):
    n = h.shape[0]
    e = src2d.shape[0] * src2d.shape[1]
    et = e // NW
    nsg = et // SCH
    sub = SCH // CHB
    rows_per_tile = n // NS

    mesh = plsc.VectorSubcoreMesh(
        core_axis_name="c", subcore_axis_name="s", num_cores=NC,
        num_subcores=NS)

    @functools.partial(
        pl.kernel,
        out_type=jax.ShapeDtypeStruct((NC, n, 128), jnp.float32),
        mesh=mesh,
        compiler_params=_sc_params(),
        scratch_types=[
            pltpu.VMEM((CHB, 512), jnp.float32),
            pltpu.VMEM((CHB, 512), jnp.float32),
            pltpu.VMEM((CHB, 128), jnp.float32),
            pltpu.VMEM((SCH // CHB, CHB), jnp.int32),
            pltpu.VMEM((SCH // CHB, CHB), jnp.int32),
            pltpu.VMEM((4 * SCH,), jnp.float32),
            pltpu.VMEM_SHARED((n, 128), jnp.float32),
            pltpu.SemaphoreType.DMA,
            pltpu.SemaphoreType.DMA,
        ],
    )
    def kern(h_hbm, al_hbm, src_hbm, dst_hbm, out_hbm,
             rows0, rows1, comb, src_c, dst_c, wa, acc_s, sem0, sem1):
        cid = lax.axis_index("c")
        sid = lax.axis_index("s")
        wid = cid * NS + sid
        base = wid * et

        @pl.loop(0, CHB)
        def _(r):
            for v in range(8):
                comb[r, pl.ds(v * LANES, LANES)] = jnp.zeros(
                    (LANES,), jnp.float32)

        zoff = sid * rows_per_tile
        nfull = rows_per_tile // CHB
        for j in range(nfull):
            pltpu.sync_copy(comb, acc_s.at[pl.ds(zoff + j * CHB, CHB)])
        rem = rows_per_tile - nfull * CHB
        if rem:
            pltpu.sync_copy(comb.at[pl.ds(0, rem)],
                            acc_s.at[pl.ds(zoff + nfull * CHB, rem)])
        plsc.subcore_barrier()

        rbufs = (rows0, rows1)
        sems = (sem0, sem1)

        def combine(rows, dsts, k):
            @pl.loop(0, CHB)
            def _(ed):
                a4 = (k * CHB + ed) * 4
                acc = [None] * 8
                for hh in range(H):
                    av = plsc.load_gather(
                        wa, [jnp.full((LANES,), a4 + hh, jnp.int32)])
                    for v in range(8):
                        seg = rows[ed, pl.ds(hh * 128 + v * LANES, LANES)]
                        t = av * seg
                        acc[v] = t if hh == 0 else acc[v] + t
                for v in range(8):
                    comb[ed, pl.ds(v * LANES, LANES)] = acc[v]

            pltpu.sync_copy(comb, acc_s.at[dsts], add=True)

        @pl.loop(0, nsg)
        def _(sg):
            soff = pl.multiple_of(base + sg * SCH, SCH)
            row0 = pl.multiple_of(soff // CHB, 8)
            pltpu.sync_copy(src_hbm.at[pl.ds(row0, sub)], src_c)
            pltpu.sync_copy(dst_hbm.at[pl.ds(row0, sub)], dst_c)
            pltpu.sync_copy(al_hbm.at[pl.ds(soff * 4, 4 * SCH)], wa)

            pltpu.async_copy(h_hbm.at[src_c.at[0]], rows0, sem0)

            for k in range(0, sub, 2):
                pltpu.async_copy(h_hbm.at[src_c.at[k + 1]], rows1, sem1)
                pltpu.make_async_copy(
                    h_hbm.at[src_c.at[k]], rows0, sem0).wait()
                combine(rows0, dst_c.at[k], k)

                if k + 2 < sub:
                    pltpu.async_copy(h_hbm.at[src_c.at[k + 2]], rows0, sem0)

                pltpu.make_async_copy(
                    h_hbm.at[src_c.at[k + 1]], rows1, sem1).wait()
                combine(rows1, dst_c.at[k + 1], k + 1)

        plsc.subcore_barrier()
        pltpu.sync_copy(
            acc_s.at[pl.ds(zoff, rows_per_tile)],
            out_hbm.at[cid, pl.ds(zoff, rows_per_tile)])

    return kern(h, al_flat, src2d, dst2d)


def _pad_rows(a, np_):
    return jnp.pad(a, ((0, np_ - a.shape[0]),) + ((0, 0),) * (a.ndim - 1))


def _amat(a_src, a_dst):
    hh, c = a_src.shape
    eye = jnp.eye(hh, dtype=jnp.float32)
    s = jnp.einsum("hc,hk->hck", a_src, eye).reshape(hh * c, hh)
    d = jnp.einsum("hc,hk->hck", a_dst, eye).reshape(hh * c, hh)
    pad = jnp.zeros((hh * c, 128 - 2 * hh), jnp.float32)
    return jnp.concatenate([s, d, pad], axis=1)


def _gat_layer(xin, edge_flat, src2d, dst2d, w, amat):
    n = xin.shape[0]
    h, alph, mrow = _tc_feats(xin, w, amat)
    as_flat = alph[:, 0:H].reshape(-1)
    ad_flat = alph[:, H:2 * H].reshape(-1)
    msum = mrow[0, 0:H] + mrow[0, H:2 * H]
    shift = jnp.maximum(msum, NEG_SLOPE * msum)
    mvec = jnp.tile(shift, LANES // H)
    w_flat, dpart = _sc_pass_a(as_flat, ad_flat, mvec, edge_flat)
    invd = _tc_invd(dpart)
    al_flat = _sc_alpha(w_flat, invd, edge_flat)
    e_pad = src2d.shape[0] * src2d.shape[1]
    al_pad = jnp.concatenate(
        [al_flat, jnp.zeros((4 * (e_pad - edge_flat.shape[0] // 2),),
                            jnp.float32)])
    return _sc_pass_b(h, al_pad, src2d, dst2d)


def kernel(x, edge_index, W1, a_src1, a_dst1, b1, W2, a_src2, a_dst2, b2):
    n = x.shape[0]
    np_ = ((n + 8 * NS - 1) // (8 * NS)) * (8 * NS)
    edge_flat = edge_index.reshape(-1)
    e = edge_index.shape[1]
    blk = NW * SCH
    e_pad = ((e + blk - 1) // blk) * blk
    zpad = jnp.zeros((e_pad - e,), jnp.int32)
    src2d = jnp.concatenate([edge_index[0], zpad]).reshape(-1, CHB)
    dst2d = jnp.concatenate([edge_index[1], zpad]).reshape(-1, CHB)
    outp1 = _gat_layer(_pad_rows(x, np_), edge_flat, src2d, dst2d, W1,
                       _amat(a_src1, a_dst1))
    hmid = _tc_mid(outp1, b1.reshape(1, -1))
    outp2 = _gat_layer(hmid, edge_flat, src2d, dst2d, W2,
                       _amat(a_src2, a_dst2))
    out = _tc_fin(outp2, b2.reshape(1, -1))
    return out[:n]
